# baseline (speedup 1.0000x reference)
import jax
import jax.numpy as jnp
from jax.experimental import pallas as pl
from jax.experimental.pallas import tpu as pltpu

B, T, D = 4, 4096, 2048
DS, DL = D // 128, 128


def _copy_body(idx_ref, x_ref, o_ref):
    o_ref[...] = x_ref[...]


def kernel(lstm_output, scalar_input):
    idx = scalar_input[:, 0].astype(jnp.int32)
    table = lstm_output.reshape(B * T, DS, DL)
    grid_spec = pltpu.PrefetchScalarGridSpec(
        num_scalar_prefetch=1,
        grid=(B,),
        in_specs=[
            pl.BlockSpec((None, DS, DL), lambda b, idx_ref: (idx_ref[b] + b * T, 0, 0)),
        ],
        out_specs=pl.BlockSpec((None, DS, DL), lambda b, idx_ref: (b, 0, 0)),
    )
    out = pl.pallas_call(
        _copy_body,
        grid_spec=grid_spec,
        out_shape=jax.ShapeDtypeStruct((B, DS, DL), jnp.float32),
    )(idx, table)
    return out.reshape(B, D)


# speedup vs baseline: 28.5123x; 28.5123x over previous
import jax
import jax.numpy as jnp
from jax.experimental import pallas as pl
from jax.experimental.pallas import tpu as pltpu

B, T, D = 4, 4096, 2048


def _gather_body(idx_ref, x_ref, o_ref, sem):
    copies = []
    for b in range(B):
        r = idx_ref[b] + b * T
        copies.append(
            pltpu.make_async_copy(x_ref.at[pl.ds(r, 1)], o_ref.at[pl.ds(b, 1)], sem)
        )
    for c in copies:
        c.start()
    for c in copies:
        c.wait()


def kernel(lstm_output, scalar_input):
    idx = scalar_input[:, 0].astype(jnp.int32)
    table = lstm_output.reshape(B * T, D)
    grid_spec = pltpu.PrefetchScalarGridSpec(
        num_scalar_prefetch=1,
        grid=(1,),
        in_specs=[pl.BlockSpec(memory_space=pl.ANY)],
        out_specs=pl.BlockSpec(memory_space=pl.ANY),
        scratch_shapes=[pltpu.SemaphoreType.DMA],
    )
    return pl.pallas_call(
        _gather_body,
        grid_spec=grid_spec,
        out_shape=jax.ShapeDtypeStruct((B, D), jnp.float32),
    )(idx, table)


# speedup vs baseline: 43.9191x; 1.5404x over previous
import jax
import jax.numpy as jnp
from jax.experimental import pallas as pl
from jax.experimental.pallas import tpu as pltpu

B, T, D = 4, 4096, 2048


def _gather_body(scal_ref, x_ref, o_ref, sem):
    copies = []
    for b in range(B):
        r = scal_ref[b].astype(jnp.int32) + b * T
        copies.append(
            pltpu.make_async_copy(x_ref.at[pl.ds(r, 1)], o_ref.at[pl.ds(b, 1)], sem)
        )
    for c in copies:
        c.start()
    for c in copies:
        c.wait()


def kernel(lstm_output, scalar_input):
    table = lstm_output.reshape(B * T, D)
    grid_spec = pltpu.PrefetchScalarGridSpec(
        num_scalar_prefetch=1,
        grid=(1,),
        in_specs=[pl.BlockSpec(memory_space=pl.ANY)],
        out_specs=pl.BlockSpec(memory_space=pl.ANY),
        scratch_shapes=[pltpu.SemaphoreType.DMA],
    )
    return pl.pallas_call(
        _gather_body,
        grid_spec=grid_spec,
        out_shape=jax.ShapeDtypeStruct((B, D), jnp.float32),
    )(scalar_input[:, 0], table)
